# baseline (speedup 1.0000x reference)
import functools
import math

import jax
import jax.numpy as jnp
from jax.experimental import pallas as pl
from jax.experimental.pallas import tpu as pltpu

N_HEADS = 16
HEAD_DIM = 128


def _matmul_bias_kernel(x_ref, w_ref, b_ref, o_ref):
    acc = jax.lax.dot_general(
        x_ref[...].astype(jnp.bfloat16),
        w_ref[...].astype(jnp.bfloat16),
        (((1,), (1,)), ((), ())),
        preferred_element_type=jnp.float32,
    ) + b_ref[...]
    o_ref[...] = acc.astype(o_ref.dtype)


def _matmul_bias(x, w, b, bn, out_dtype):
    t, k = x.shape
    n = w.shape[0]
    grid = (n // bn,)
    return pl.pallas_call(
        _matmul_bias_kernel,
        grid=grid,
        in_specs=[
            pl.BlockSpec((t, k), lambda j: (0, 0)),
            pl.BlockSpec((bn, k), lambda j: (j, 0)),
            pl.BlockSpec((1, bn), lambda j: (0, j)),
        ],
        out_specs=pl.BlockSpec((t, bn), lambda j: (0, j)),
        out_shape=jax.ShapeDtypeStruct((t, n), out_dtype),
        compiler_params=pltpu.CompilerParams(
            dimension_semantics=("parallel",),
        ),
    )(x, w, b.reshape(1, n))


def _flash_kernel(q_ref, k_ref, v_ref, o_ref, acc_ref, m_ref, l_ref,
                  *, bq, bk, scale):
    i = pl.program_id(1)
    m_ref[...] = jnp.full_like(m_ref, -1e30)
    l_ref[...] = jnp.zeros_like(l_ref)
    acc_ref[...] = jnp.zeros_like(acc_ref)

    q = q_ref[...]
    row_base = i * bq
    nchunks = (row_base + bq + bk - 1) // bk

    def body(j, _):
        kc = k_ref[pl.ds(j * bk, bk), :]
        s = jax.lax.dot_general(
            q, kc, (((1,), (1,)), ((), ())),
            preferred_element_type=jnp.float32,
        ) * scale
        rows = row_base + jax.lax.broadcasted_iota(jnp.int32, (bq, bk), 0)
        cols = j * bk + jax.lax.broadcasted_iota(jnp.int32, (bq, bk), 1)
        s = jnp.where(rows >= cols, s, -1e30)

        m_prev = m_ref[...]
        m_new = jnp.maximum(m_prev, jnp.max(s, axis=1, keepdims=True))
        alpha = jnp.exp(m_prev - m_new)
        p = jnp.exp(s - m_new)
        l_ref[...] = l_ref[...] * alpha + jnp.sum(p, axis=1, keepdims=True)
        vc = v_ref[pl.ds(j * bk, bk), :]
        pv = jax.lax.dot_general(
            p.astype(jnp.bfloat16), vc, (((1,), (0,)), ((), ())),
            preferred_element_type=jnp.float32,
        )
        acc_ref[...] = acc_ref[...] * alpha + pv
        m_ref[...] = m_new
        return 0

    jax.lax.fori_loop(0, nchunks, body, 0)
    o_ref[...] = (acc_ref[...] / l_ref[...]).astype(o_ref.dtype)


def _flash_attention(qkv, t, c, bq, bk):
    h = N_HEADS
    hs = HEAD_DIM
    nq = t // bq
    hb = c // hs
    scale = 1.0 / math.sqrt(hs)
    kern = functools.partial(_flash_kernel, bq=bq, bk=bk, scale=scale)
    return pl.pallas_call(
        kern,
        grid=(h, nq),
        in_specs=[
            pl.BlockSpec((bq, hs), lambda hh, i: (i, hh)),
            pl.BlockSpec((t, hs), lambda hh, i: (0, hb + hh)),
            pl.BlockSpec((t, hs), lambda hh, i: (0, 2 * hb + hh)),
        ],
        out_specs=pl.BlockSpec((bq, hs), lambda hh, i: (i, hh)),
        out_shape=jax.ShapeDtypeStruct((t, c), jnp.bfloat16),
        scratch_shapes=[
            pltpu.VMEM((bq, hs), jnp.float32),
            pltpu.VMEM((bq, 1), jnp.float32),
            pltpu.VMEM((bq, 1), jnp.float32),
        ],
        compiler_params=pltpu.CompilerParams(
            dimension_semantics=("parallel", "arbitrary"),
        ),
    )(qkv, qkv, qkv)


@jax.jit
def _attention_impl(x, W_attn, b_attn, W_proj, b_proj):
    b, t, c = x.shape
    x2 = x.reshape(t, c)
    qkv = _matmul_bias(x2, W_attn, b_attn, bn=512, out_dtype=jnp.bfloat16)
    y = _flash_attention(qkv, t, c, bq=256, bk=256)
    out = _matmul_bias(y, W_proj, b_proj, bn=512, out_dtype=jnp.float32)
    return out.reshape(b, t, c)


def kernel(x, W_attn, b_attn, W_proj, b_proj):
    return _attention_impl(x, W_attn, b_attn, W_proj, b_proj)


# speedup vs baseline: 1.3180x; 1.3180x over previous
import functools
import math

import jax
import jax.numpy as jnp
from jax.experimental import pallas as pl
from jax.experimental.pallas import tpu as pltpu

N_HEADS = 16
HEAD_DIM = 128


def _matmul_bias_kernel(x_ref, w_ref, b_ref, o_ref):
    acc = jax.lax.dot_general(
        x_ref[...].astype(jnp.bfloat16),
        w_ref[...].astype(jnp.bfloat16),
        (((1,), (1,)), ((), ())),
        preferred_element_type=jnp.float32,
    ) + b_ref[...]
    o_ref[...] = acc.astype(o_ref.dtype)


def _matmul_bias(x, w, b, bn, out_dtype):
    t, k = x.shape
    n = w.shape[0]
    grid = (n // bn,)
    return pl.pallas_call(
        _matmul_bias_kernel,
        grid=grid,
        in_specs=[
            pl.BlockSpec((t, k), lambda j: (0, 0)),
            pl.BlockSpec((bn, k), lambda j: (j, 0)),
            pl.BlockSpec((1, bn), lambda j: (0, j)),
        ],
        out_specs=pl.BlockSpec((t, bn), lambda j: (0, j)),
        out_shape=jax.ShapeDtypeStruct((t, n), out_dtype),
        compiler_params=pltpu.CompilerParams(
            dimension_semantics=("parallel",),
        ),
    )(x, w, b.reshape(1, n))


def _flash_kernel(q_ref, k_ref, v_ref, o_ref, acc_ref, m_ref, l_ref,
                  *, bq, bk, scale):
    i = pl.program_id(1)
    row_base = i * bq
    q = q_ref[...]

    def chunk(j, masked):
        kc = k_ref[pl.ds(j * bk, bk), :]
        s = jax.lax.dot_general(
            q, kc, (((1,), (1,)), ((), ())),
            preferred_element_type=jnp.float32,
        ) * scale
        if masked:
            rows = jax.lax.broadcasted_iota(jnp.int32, (bq, bk), 0)
            cols = jax.lax.broadcasted_iota(jnp.int32, (bq, bk), 1)
            s = jnp.where(rows >= cols, s, -1e30)

        m_prev = m_ref[...]
        m_new = jnp.maximum(m_prev, jnp.max(s, axis=1, keepdims=True))
        alpha = jnp.exp(m_prev - m_new)
        p = jnp.exp(s - m_new)
        l_ref[...] = l_ref[...] * alpha + jnp.sum(p, axis=1, keepdims=True)
        vc = v_ref[pl.ds(j * bk, bk), :]
        pv = jax.lax.dot_general(
            p.astype(jnp.bfloat16), vc, (((1,), (0,)), ((), ())),
            preferred_element_type=jnp.float32,
        )
        acc_ref[...] = acc_ref[...] * alpha + pv
        m_ref[...] = m_new

    m_ref[...] = jnp.full_like(m_ref, -1e30)
    l_ref[...] = jnp.zeros_like(l_ref)
    acc_ref[...] = jnp.zeros_like(acc_ref)

    def body(j, _):
        chunk(j, masked=False)
        return 0

    jax.lax.fori_loop(0, i, body, 0)
    chunk(i, masked=True)
    o_ref[...] = (acc_ref[...] / l_ref[...]).astype(o_ref.dtype)


def _flash_attention(qkv, t, c, bq, bk):
    h = N_HEADS
    hs = HEAD_DIM
    nq = t // bq
    hb = c // hs
    scale = 1.0 / math.sqrt(hs)
    kern = functools.partial(_flash_kernel, bq=bq, bk=bk, scale=scale)
    return pl.pallas_call(
        kern,
        grid=(h, nq),
        in_specs=[
            pl.BlockSpec((bq, hs), lambda hh, i: (i, hh)),
            pl.BlockSpec((t, hs), lambda hh, i: (0, hb + hh)),
            pl.BlockSpec((t, hs), lambda hh, i: (0, 2 * hb + hh)),
        ],
        out_specs=pl.BlockSpec((bq, hs), lambda hh, i: (i, hh)),
        out_shape=jax.ShapeDtypeStruct((t, c), jnp.bfloat16),
        scratch_shapes=[
            pltpu.VMEM((bq, hs), jnp.float32),
            pltpu.VMEM((bq, 1), jnp.float32),
            pltpu.VMEM((bq, 1), jnp.float32),
        ],
        compiler_params=pltpu.CompilerParams(
            dimension_semantics=("parallel", "arbitrary"),
        ),
    )(qkv, qkv, qkv)


@jax.jit
def _attention_impl(x, W_attn, b_attn, W_proj, b_proj):
    b, t, c = x.shape
    x2 = x.reshape(t, c).astype(jnp.bfloat16)
    qkv = _matmul_bias(x2, W_attn, b_attn, bn=512, out_dtype=jnp.bfloat16)
    y = _flash_attention(qkv, t, c, bq=512, bk=512)
    out = _matmul_bias(y, W_proj, b_proj, bn=512, out_dtype=jnp.float32)
    return out.reshape(b, t, c)


def kernel(x, W_attn, b_attn, W_proj, b_proj):
    return _attention_impl(x, W_attn, b_attn, W_proj, b_proj)


# speedup vs baseline: 2.0488x; 1.5545x over previous
import functools
import math

import jax
import jax.numpy as jnp
from jax.experimental import pallas as pl
from jax.experimental.pallas import tpu as pltpu

N_HEADS = 16
HEAD_DIM = 128


def _matmul_bias_kernel(x_ref, w_ref, b_ref, o_ref):
    acc = jax.lax.dot_general(
        x_ref[...].astype(jnp.bfloat16),
        w_ref[...].astype(jnp.bfloat16),
        (((1,), (1,)), ((), ())),
        preferred_element_type=jnp.float32,
    ) + b_ref[...]
    o_ref[...] = acc.astype(o_ref.dtype)


def _matmul_bias(x, w, b, bn, out_dtype):
    t, k = x.shape
    n = w.shape[0]
    grid = (n // bn,)
    return pl.pallas_call(
        _matmul_bias_kernel,
        grid=grid,
        in_specs=[
            pl.BlockSpec((t, k), lambda j: (0, 0)),
            pl.BlockSpec((bn, k), lambda j: (j, 0)),
            pl.BlockSpec((1, bn), lambda j: (0, j)),
        ],
        out_specs=pl.BlockSpec((t, bn), lambda j: (0, j)),
        out_shape=jax.ShapeDtypeStruct((t, n), out_dtype),
        compiler_params=pltpu.CompilerParams(
            dimension_semantics=("parallel",),
        ),
    )(x, w, b.reshape(1, n))


def _flash_kernel(q_ref, k_ref, v_ref, o_ref, acc_ref, l_ref, kmax_ref,
                  *, bq, bk, scale):
    i = pl.program_id(1)

    @pl.when(i == 0)
    def _():
        kf = k_ref[...].astype(jnp.float32)
        kn = jnp.sum(kf * kf, axis=1, keepdims=True)
        kmax_ref[0] = jnp.sqrt(jnp.max(kn))

    q = q_ref[...]
    qf = q.astype(jnp.float32)
    qn = jnp.sqrt(jnp.sum(qf * qf, axis=1, keepdims=True))
    m_r = qn * (kmax_ref[0] * scale)

    acc_ref[...] = jnp.zeros_like(acc_ref)
    l_ref[...] = jnp.zeros_like(l_ref)
    ones_bk = jnp.ones((bk, 128), jnp.bfloat16)

    def chunk(j, masked):
        kc = k_ref[pl.ds(j * bk, bk), :]
        s = jax.lax.dot_general(
            q, kc, (((1,), (1,)), ((), ())),
            preferred_element_type=jnp.float32,
        )
        p = jnp.exp(s * scale - m_r)
        if masked:
            rows = jax.lax.broadcasted_iota(jnp.int32, (bq, bk), 0)
            cols = jax.lax.broadcasted_iota(jnp.int32, (bq, bk), 1)
            p = jnp.where(rows >= cols, p, 0.0)
        pb = p.astype(jnp.bfloat16)
        vc = v_ref[pl.ds(j * bk, bk), :]
        pv = jax.lax.dot_general(
            pb, vc, (((1,), (0,)), ((), ())),
            preferred_element_type=jnp.float32,
        )
        ps = jax.lax.dot_general(
            pb, ones_bk, (((1,), (0,)), ((), ())),
            preferred_element_type=jnp.float32,
        )
        acc_ref[...] += pv
        l_ref[...] += ps

    def body(j, _):
        chunk(j, masked=False)
        return 0

    jax.lax.fori_loop(0, i, body, 0)
    chunk(i, masked=True)
    o_ref[...] = (acc_ref[...] / l_ref[...]).astype(o_ref.dtype)


def _flash_attention(qkv, t, c, bq, bk):
    h = N_HEADS
    hs = HEAD_DIM
    nq = t // bq
    hb = c // hs
    scale = 1.0 / math.sqrt(hs)
    kern = functools.partial(_flash_kernel, bq=bq, bk=bk, scale=scale)
    return pl.pallas_call(
        kern,
        grid=(h, nq),
        in_specs=[
            pl.BlockSpec((bq, hs), lambda hh, i: (i, hh)),
            pl.BlockSpec((t, hs), lambda hh, i: (0, hb + hh)),
            pl.BlockSpec((t, hs), lambda hh, i: (0, 2 * hb + hh)),
        ],
        out_specs=pl.BlockSpec((bq, hs), lambda hh, i: (i, hh)),
        out_shape=jax.ShapeDtypeStruct((t, c), jnp.bfloat16),
        scratch_shapes=[
            pltpu.VMEM((bq, hs), jnp.float32),
            pltpu.VMEM((bq, hs), jnp.float32),
            pltpu.SMEM((1,), jnp.float32),
        ],
        compiler_params=pltpu.CompilerParams(
            dimension_semantics=("parallel", "arbitrary"),
        ),
    )(qkv, qkv, qkv)


@jax.jit
def _attention_impl(x, W_attn, b_attn, W_proj, b_proj):
    b, t, c = x.shape
    x2 = x.reshape(t, c).astype(jnp.bfloat16)
    qkv = _matmul_bias(x2, W_attn, b_attn, bn=512, out_dtype=jnp.bfloat16)
    y = _flash_attention(qkv, t, c, bq=512, bk=512)
    out = _matmul_bias(y, W_proj, b_proj, bn=512, out_dtype=jnp.float32)
    return out.reshape(b, t, c)


def kernel(x, W_attn, b_attn, W_proj, b_proj):
    return _attention_impl(x, W_attn, b_attn, W_proj, b_proj)


# speedup vs baseline: 2.1269x; 1.0381x over previous
import functools
import math

import jax
import jax.numpy as jnp
from jax.experimental import pallas as pl
from jax.experimental.pallas import tpu as pltpu

N_HEADS = 16
HEAD_DIM = 128


def _matmul_bias_kernel(x_ref, w_ref, b_ref, o_ref):
    acc = jax.lax.dot_general(
        x_ref[...].astype(jnp.bfloat16),
        w_ref[...].astype(jnp.bfloat16),
        (((1,), (1,)), ((), ())),
        preferred_element_type=jnp.float32,
    ) + b_ref[...]
    o_ref[...] = acc.astype(o_ref.dtype)


def _matmul_bias(x, w, b, bn, out_dtype):
    t, k = x.shape
    n = w.shape[0]
    grid = (n // bn,)
    return pl.pallas_call(
        _matmul_bias_kernel,
        grid=grid,
        in_specs=[
            pl.BlockSpec((t, k), lambda j: (0, 0)),
            pl.BlockSpec((bn, k), lambda j: (j, 0)),
            pl.BlockSpec((1, bn), lambda j: (0, j)),
        ],
        out_specs=pl.BlockSpec((t, bn), lambda j: (0, j)),
        out_shape=jax.ShapeDtypeStruct((t, n), out_dtype),
        compiler_params=pltpu.CompilerParams(
            dimension_semantics=("parallel",),
        ),
    )(x, w, b.reshape(1, n))


def _flash_kernel(q_ref, qc_ref, k_ref, v_ref, o_ref, acc_ref, vaug_ref,
                  m_ref, *, bq, bk, scale):
    i = pl.program_id(1)
    hs = HEAD_DIM

    @pl.when(i == 0)
    def _():
        qf = qc_ref[...].astype(jnp.float32)
        qn = jnp.sum(qf * qf, axis=1, keepdims=True)
        kf = k_ref[...].astype(jnp.float32)
        kn = jnp.sum(kf * kf, axis=1, keepdims=True)
        m_ref[0] = jnp.sqrt(jnp.max(qn)) * jnp.sqrt(jnp.max(kn)) * scale
        vaug_ref[:, :hs] = v_ref[...]
        vaug_ref[:, hs:] = jnp.ones_like(vaug_ref[:, hs:])

    q = q_ref[...]
    m_r = m_ref[0]
    acc_ref[...] = jnp.zeros_like(acc_ref)

    def chunk(j, masked):
        kc = k_ref[pl.ds(j * bk, bk), :]
        s = jax.lax.dot_general(
            q, kc, (((1,), (1,)), ((), ())),
            preferred_element_type=jnp.float32,
        )
        p = jnp.exp(s * scale - m_r)
        if masked:
            rows = jax.lax.broadcasted_iota(jnp.int32, (bq, bk), 0)
            cols = jax.lax.broadcasted_iota(jnp.int32, (bq, bk), 1)
            p = jnp.where(rows >= cols, p, 0.0)
        pb = p.astype(jnp.bfloat16)
        vc = vaug_ref[pl.ds(j * bk, bk), :]
        acc_ref[...] += jax.lax.dot_general(
            pb, vc, (((1,), (0,)), ((), ())),
            preferred_element_type=jnp.float32,
        )

    def body(j, _):
        chunk(j, masked=False)
        return 0

    jax.lax.fori_loop(0, i, body, 0)
    chunk(i, masked=True)
    o_ref[...] = (acc_ref[:, :hs] / acc_ref[:, hs:]).astype(o_ref.dtype)


def _flash_attention(qkv, t, c, bq, bk):
    h = N_HEADS
    hs = HEAD_DIM
    nq = t // bq
    hb = c // hs
    scale = 1.0 / math.sqrt(hs)
    kern = functools.partial(_flash_kernel, bq=bq, bk=bk, scale=scale)
    return pl.pallas_call(
        kern,
        grid=(h, nq),
        in_specs=[
            pl.BlockSpec((bq, hs), lambda hh, i: (i, hh)),
            pl.BlockSpec((t, hs), lambda hh, i: (0, hh)),
            pl.BlockSpec((t, hs), lambda hh, i: (0, hb + hh)),
            pl.BlockSpec((t, hs), lambda hh, i: (0, 2 * hb + hh)),
        ],
        out_specs=pl.BlockSpec((bq, hs), lambda hh, i: (i, hh)),
        out_shape=jax.ShapeDtypeStruct((t, c), jnp.bfloat16),
        scratch_shapes=[
            pltpu.VMEM((bq, 2 * hs), jnp.float32),
            pltpu.VMEM((t, 2 * hs), jnp.bfloat16),
            pltpu.SMEM((1,), jnp.float32),
        ],
        compiler_params=pltpu.CompilerParams(
            dimension_semantics=("parallel", "arbitrary"),
        ),
    )(qkv, qkv, qkv, qkv)


@jax.jit
def _attention_impl(x, W_attn, b_attn, W_proj, b_proj):
    b, t, c = x.shape
    x2 = x.reshape(t, c).astype(jnp.bfloat16)
    qkv = _matmul_bias(x2, W_attn, b_attn, bn=512, out_dtype=jnp.bfloat16)
    y = _flash_attention(qkv, t, c, bq=512, bk=512)
    out = _matmul_bias(y, W_proj, b_proj, bn=512, out_dtype=jnp.float32)
    return out.reshape(b, t, c)


def kernel(x, W_attn, b_attn, W_proj, b_proj):
    return _attention_impl(x, W_attn, b_attn, W_proj, b_proj)
